# baseline (device time: 46932 ns/iter reference)
import jax
import jax.numpy as jnp
from jax import lax
from jax.experimental import pallas as pl
from jax.experimental.pallas import tpu as pltpu

N_DEV = 8


def kernel(x, w_mat):
    m_per, k = x.shape
    _, n = w_mat.shape
    n_per = n // N_DEV

    def body(x_ref, w_ref, out_ref, send_buf, recv_buf, send_sems, recv_sems):
        my = lax.axis_index("i")

        barrier_sem = pltpu.get_barrier_semaphore()
        pl.semaphore_signal(barrier_sem, inc=1)

        xb = x_ref[:, :].astype(jnp.bfloat16)

        def block(tgt):
            wb = w_ref[:, pl.ds(tgt * n_per, n_per)].astype(jnp.bfloat16)
            y = jnp.dot(xb, wb, preferred_element_type=jnp.float32)
            return y * jax.nn.sigmoid(y)

        for s in range(1, N_DEV):
            tgt = lax.rem(my + s, N_DEV)
            send_buf[s - 1, :, :] = block(tgt).astype(jnp.bfloat16)

        pl.semaphore_wait(barrier_sem, 1)

        rdmas = []
        for s in range(1, N_DEV):
            tgt = lax.rem(my + s, N_DEV)
            rdma = pltpu.make_async_remote_copy(
                src_ref=send_buf.at[s - 1],
                dst_ref=recv_buf.at[s - 1],
                send_sem=send_sems.at[s - 1],
                recv_sem=recv_sems.at[s - 1],
                device_id=(tgt,),
                device_id_type=pl.DeviceIdType.MESH,
            )
            rdma.start()
            rdmas.append(rdma)

        out_ref[pl.ds(my * m_per, m_per), :] = block(my)

        for s in range(1, N_DEV):
            src = lax.rem(my - s + N_DEV, N_DEV)
            rdmas[s - 1].wait_send()
            rdmas[s - 1].wait_recv()
            out_ref[pl.ds(src * m_per, m_per), :] = recv_buf[s - 1, :, :].astype(
                jnp.float32
            )

    return pl.pallas_call(
        body,
        out_shape=jax.ShapeDtypeStruct((N_DEV * m_per, n_per), jnp.float32),
        in_specs=[
            pl.BlockSpec(memory_space=pltpu.VMEM),
            pl.BlockSpec(memory_space=pltpu.VMEM),
        ],
        out_specs=pl.BlockSpec(memory_space=pltpu.VMEM),
        scratch_shapes=[
            pltpu.VMEM((N_DEV - 1, m_per, n_per), jnp.bfloat16),
            pltpu.VMEM((N_DEV - 1, m_per, n_per), jnp.bfloat16),
            pltpu.SemaphoreType.DMA((N_DEV - 1,)),
            pltpu.SemaphoreType.DMA((N_DEV - 1,)),
        ],
        compiler_params=pltpu.CompilerParams(
            vmem_limit_bytes=100 * 1024 * 1024,
            collective_id=0,
        ),
    )(x, w_mat)


# device time: 43693 ns/iter; 1.0741x vs baseline; 1.0741x over previous
import jax
import jax.numpy as jnp
from jax import lax
from jax.experimental import pallas as pl
from jax.experimental.pallas import tpu as pltpu

N_DEV = 8


def kernel(x, w_mat):
    m_per, k = x.shape
    _, n = w_mat.shape
    n_per = n // N_DEV

    def body(x_ref, w_ref, out_ref, send_buf, recv_buf, send_sems, recv_sems):
        my = lax.axis_index("i")

        barrier_sem = pltpu.get_barrier_semaphore()
        pl.semaphore_signal(barrier_sem, inc=1)

        pl.semaphore_wait(barrier_sem, 1)

        xb = x_ref[:, :].astype(jnp.bfloat16)

        def block(tgt):
            wb = w_ref[:, pl.ds(tgt * n_per, n_per)].astype(jnp.bfloat16)
            y = jnp.dot(xb, wb, preferred_element_type=jnp.float32)
            return y * jax.nn.sigmoid(y)

        h = lax.div(my, 4)
        p = lax.rem(my, 4)
        oh = (1 - h) * 4

        def tgt_of(s):
            if s <= 4:
                return oh + lax.rem(p + s - 1, 4)
            return h * 4 + lax.rem(p + s - 4, 4)

        def src_of(s):
            if s <= 4:
                return oh + lax.rem(p - s + 1 + 4, 4)
            return h * 4 + lax.rem(p - s + 4 + 4, 4)

        rdmas = []
        for s in range(1, N_DEV):
            send_buf[s - 1, :, :] = block(tgt_of(s)).astype(jnp.bfloat16)
            rdma = pltpu.make_async_remote_copy(
                src_ref=send_buf.at[s - 1],
                dst_ref=recv_buf.at[s - 1],
                send_sem=send_sems.at[s - 1],
                recv_sem=recv_sems.at[s - 1],
                device_id=(tgt_of(s),),
                device_id_type=pl.DeviceIdType.MESH,
            )
            rdma.start()
            rdmas.append(rdma)

        out_ref[pl.ds(my * m_per, m_per), :] = block(my)

        for s in range(1, N_DEV):
            rdmas[s - 1].wait_send()
            rdmas[s - 1].wait_recv()
            out_ref[pl.ds(src_of(s) * m_per, m_per), :] = recv_buf[
                s - 1, :, :
            ].astype(jnp.float32)

    return pl.pallas_call(
        body,
        out_shape=jax.ShapeDtypeStruct((N_DEV * m_per, n_per), jnp.float32),
        in_specs=[
            pl.BlockSpec(memory_space=pltpu.VMEM),
            pl.BlockSpec(memory_space=pltpu.VMEM),
        ],
        out_specs=pl.BlockSpec(memory_space=pltpu.VMEM),
        scratch_shapes=[
            pltpu.VMEM((N_DEV - 1, m_per, n_per), jnp.bfloat16),
            pltpu.VMEM((N_DEV - 1, m_per, n_per), jnp.bfloat16),
            pltpu.SemaphoreType.DMA((N_DEV - 1,)),
            pltpu.SemaphoreType.DMA((N_DEV - 1,)),
        ],
        compiler_params=pltpu.CompilerParams(
            vmem_limit_bytes=100 * 1024 * 1024,
            collective_id=0,
        ),
    )(x, w_mat)


# device time: 35432 ns/iter; 1.3246x vs baseline; 1.2332x over previous
import jax
import jax.numpy as jnp
from jax import lax
from jax.experimental import pallas as pl
from jax.experimental.pallas import tpu as pltpu

N_DEV = 8


def kernel(x, w_mat):
    m_per, k = x.shape
    _, n = w_mat.shape
    n_per = n // N_DEV

    def body(
        x_ref, w_ref, out_ref, send_buf, recv_buf, w_tile, stage,
        send_sems, recv_sems, fetch_sems, out_sems,
    ):
        pending = [None, None]

        def store_rows(row_start, slot, values):
            if pending[slot] is not None:
                pending[slot].wait()
            stage[slot, :, :] = values
            cp = pltpu.make_async_copy(
                stage.at[slot],
                out_ref.at[pl.ds(row_start, m_per), :],
                out_sems.at[slot],
            )
            cp.start()
            pending[slot] = cp
        my = lax.axis_index("i")

        barrier_sem = pltpu.get_barrier_semaphore()
        for s in range(1, N_DEV):
            pl.semaphore_signal(
                barrier_sem, inc=1,
                device_id=(lax.rem(my + s, N_DEV),),
                device_id_type=pl.DeviceIdType.MESH,
            )

        xb = x_ref[:, :].astype(jnp.bfloat16)

        def fetch(tgt, slot):
            cp = pltpu.make_async_copy(
                w_ref.at[:, pl.ds(tgt * n_per, n_per)],
                w_tile.at[slot],
                fetch_sems.at[slot],
            )
            cp.start()
            return cp

        def block(slot):
            wb = w_tile[slot, :, :].astype(jnp.bfloat16)
            y = jnp.dot(xb, wb, preferred_element_type=jnp.float32)
            return y * jax.nn.sigmoid(y)

        h = lax.div(my, 4)
        p = lax.rem(my, 4)
        oh = (1 - h) * 4

        def tgt_of(s):
            if s <= 4:
                return oh + lax.rem(p + s - 1, 4)
            return h * 4 + lax.rem(p + s - 4, 4)

        def src_of(s):
            if s <= 4:
                return oh + lax.rem(p - s + 1 + 4, 4)
            return h * 4 + lax.rem(p - s + 4 + 4, 4)

        fetches = [fetch(tgt_of(1), 0)]
        rdmas = []
        for s in range(1, N_DEV):
            slot = (s - 1) % 2
            fetches.append(fetch(tgt_of(s + 1) if s + 1 < N_DEV else my, 1 - slot))
            fetches[s - 1].wait()
            send_buf[s - 1, :, :] = block(slot).astype(jnp.bfloat16)
            rdma = pltpu.make_async_remote_copy(
                src_ref=send_buf.at[s - 1],
                dst_ref=recv_buf.at[s - 1],
                send_sem=send_sems.at[s - 1],
                recv_sem=recv_sems.at[s - 1],
                device_id=(tgt_of(s),),
                device_id_type=pl.DeviceIdType.MESH,
            )
            if s == 1:
                pl.semaphore_wait(barrier_sem, N_DEV - 1)
            rdma.start()
            rdmas.append(rdma)

        fetches[N_DEV - 1].wait()
        store_rows(my * m_per, 0, block((N_DEV - 1) % 2))

        for s in range(1, N_DEV):
            rdmas[s - 1].wait_send()
            rdmas[s - 1].wait_recv()
            store_rows(
                src_of(s) * m_per, s % 2,
                recv_buf[s - 1, :, :].astype(jnp.float32),
            )
        pending[0].wait()
        pending[1].wait()

    return pl.pallas_call(
        body,
        out_shape=jax.ShapeDtypeStruct((N_DEV * m_per, n_per), jnp.float32),
        in_specs=[
            pl.BlockSpec(memory_space=pltpu.VMEM),
            pl.BlockSpec(memory_space=pltpu.MemorySpace.HBM),
        ],
        out_specs=pl.BlockSpec(memory_space=pltpu.MemorySpace.HBM),
        scratch_shapes=[
            pltpu.VMEM((N_DEV - 1, m_per, n_per), jnp.bfloat16),
            pltpu.VMEM((N_DEV - 1, m_per, n_per), jnp.bfloat16),
            pltpu.VMEM((2, k, n_per), jnp.float32),
            pltpu.VMEM((2, m_per, n_per), jnp.float32),
            pltpu.SemaphoreType.DMA((N_DEV - 1,)),
            pltpu.SemaphoreType.DMA((N_DEV - 1,)),
            pltpu.SemaphoreType.DMA((2,)),
            pltpu.SemaphoreType.DMA((2,)),
        ],
        compiler_params=pltpu.CompilerParams(
            vmem_limit_bytes=100 * 1024 * 1024,
            collective_id=0,
        ),
    )(x, w_mat)
